# Initial kernel scaffold; baseline (speedup 1.0000x reference)
import jax
import jax.numpy as jnp
from jax.experimental import pallas as pl


def kernel(x, edge_index, batch, params):
    raise NotImplementedError("write your pallas kernel here")


# speedup vs baseline: 4.6806x; 4.6806x over previous
import functools

import jax
import jax.numpy as jnp
from jax import lax
from jax.experimental import pallas as pl
from jax.experimental.pallas import tpu as pltpu
from jax.experimental.pallas import tpu_sc as plsc

NN = 10000
EE = 320000
DD = 128
NG = 16
EPS = 1e-5

NPAD = 10240
NW = 32
EPT = 10240
CH = 128
NCH = EPT // CH
EPAD = NW * EPT
EALLOC = EPAD + CH
ROWS_PT = NPAD // 16
DEGW = 16

_mesh = plsc.VectorSubcoreMesh(core_axis_name="c", subcore_axis_name="s")


@functools.partial(
    pl.kernel,
    out_type=jax.ShapeDtypeStruct((2, NPAD, DD), jnp.float32),
    mesh=_mesh,
    scratch_types=[
        pltpu.VMEM((CH,), jnp.int32),
        pltpu.VMEM((CH,), jnp.int32),
        pltpu.VMEM((CH,), jnp.int32),
        pltpu.VMEM((CH,), jnp.int32),
        pltpu.VMEM((CH, DD), jnp.float32),
        pltpu.VMEM((CH, DD), jnp.float32),
        pltpu.VMEM_SHARED((NPAD, DD), jnp.float32),
        pltpu.SemaphoreType.DMA,
        pltpu.SemaphoreType.DMA,
    ],
)
def _agg(g_hbm, src_hbm, dst_hbm, out_hbm,
         sidx0, sidx1, didx0, didx1, rows0, rows1, acc, sem0, sem1):
    cid = lax.axis_index("c")
    sid = lax.axis_index("s")
    wid = cid * 16 + sid
    ebase = wid * EPT

    zero = jnp.zeros((16,), jnp.float32)

    def _zrow(i, _):
        for j in range(DD // 16):
            rows0[i, pl.ds(j * 16, 16)] = zero
        return 0

    lax.fori_loop(0, CH, _zrow, 0)
    rbase = sid * ROWS_PT
    for r in range(ROWS_PT // CH):
        pltpu.sync_copy(rows0, acc.at[pl.ds(rbase + r * CH, CH)])
    plsc.subcore_barrier()

    def _fire(k, sidx, didx, rows, sem):
        off = ebase + k * CH
        pltpu.sync_copy(src_hbm.at[pl.ds(off, CH)], sidx)
        pltpu.sync_copy(dst_hbm.at[pl.ds(off, CH)], didx)
        pltpu.async_copy(g_hbm.at[sidx], rows, sem)

    _fire(0, sidx0, didx0, rows0, sem0)

    def _body(t, _):
        pltpu.make_async_copy(g_hbm.at[sidx0], rows0, sem0).wait()
        _fire(2 * t + 1, sidx1, didx1, rows1, sem1)
        pltpu.sync_copy(rows0, acc.at[didx0], add=True)
        pltpu.make_async_copy(g_hbm.at[sidx1], rows1, sem1).wait()
        _fire(2 * t + 2, sidx0, didx0, rows0, sem0)
        pltpu.sync_copy(rows1, acc.at[didx1], add=True)
        return 0

    lax.fori_loop(0, NCH // 2, _body, 0)
    pltpu.make_async_copy(g_hbm.at[sidx0], rows0, sem0).wait()
    plsc.subcore_barrier()

    for r in range(ROWS_PT // CH):
        sl = pl.ds(rbase + r * CH, CH)
        pltpu.sync_copy(acc.at[sl], out_hbm.at[cid, sl])


def _mm_body(x_ref, w_ref, o_ref):
    o_ref[...] = jnp.dot(x_ref[...], w_ref[...],
                         preferred_element_type=jnp.float32)


_mm = pl.pallas_call(
    _mm_body,
    out_shape=jax.ShapeDtypeStruct((NPAD, DD), jnp.float32),
)


def _scale_body(d0_ref, d1_ref, m_ref, g_ref, dinv_ref):
    deg = d0_ref[...] + d1_ref[...] + 1.0
    dinv = lax.rsqrt(deg)
    dinv_ref[...] = dinv
    g_ref[...] = dinv * m_ref[...]


_scale = pl.pallas_call(
    _scale_body,
    out_shape=(
        jax.ShapeDtypeStruct((NPAD, DD), jnp.float32),
        jax.ShapeDtypeStruct((NPAD, 1), jnp.float32),
    ),
)


def _norm_body(p0_ref, p1_ref, g_ref, dinv_ref, b_ref,
               al_ref, ga_ref, be_ref, w_ref, o_ref):
    dinv = dinv_ref[...]
    h = dinv * (p0_ref[...] + p1_ref[...] + g_ref[...]) + b_ref[...]
    rmask = lax.broadcasted_iota(jnp.int32, (NPAD, 1), 0) < NN
    h = jnp.where(rmask, h, 0.0)
    mean = jnp.sum(h, axis=0, keepdims=True) * (1.0 / NN)
    o = h - al_ref[...] * mean
    o = jnp.where(rmask, o, 0.0)
    var = jnp.sum(o * o, axis=0, keepdims=True) * (1.0 / NN)
    a = ga_ref[...] * (o * lax.rsqrt(var + EPS)) + be_ref[...]
    a = jnp.where(rmask, jnp.maximum(a, 0.0), 0.0)
    o_ref[...] = dinv * jnp.dot(a, w_ref[...],
                                preferred_element_type=jnp.float32)


_norm = pl.pallas_call(
    _norm_body,
    out_shape=jax.ShapeDtypeStruct((NPAD, DD), jnp.float32),
)


def _head_body(p0_ref, p1_ref, g_ref, dinv_ref, b_ref, batch_ref,
               wh1_ref, bh1_ref, wh2_ref, bh2_ref, o_ref):
    h = dinv_ref[...] * (p0_ref[...] + p1_ref[...] + g_ref[...]) + b_ref[...]
    gid = lax.broadcasted_iota(jnp.int32, (NG, NPAD), 0)
    oh = (gid == batch_ref[...]).astype(jnp.float32)
    sums = jnp.dot(oh, h, preferred_element_type=jnp.float32)
    cnt = jnp.sum(oh, axis=1, keepdims=True)
    pooled = sums / jnp.maximum(cnt, 1.0)
    z = jnp.maximum(
        jnp.dot(pooled, wh1_ref[...], preferred_element_type=jnp.float32)
        + bh1_ref[...], 0.0)
    o_ref[...] = jnp.dot(z, wh2_ref[...],
                         preferred_element_type=jnp.float32) + bh2_ref[...]


def _make_head(nhid, nout):
    return pl.pallas_call(
        _head_body,
        out_shape=jax.ShapeDtypeStruct((NG, nout), jnp.float32),
    )


@jax.jit
def kernel(x, edge_index, batch, params):
    src = edge_index[0].astype(jnp.int32)
    dst = edge_index[1].astype(jnp.int32)
    pad = jnp.full((EALLOC - EE,), NN, dtype=jnp.int32)
    srcp = jnp.concatenate([src, pad])
    dstp = jnp.concatenate([dst, pad])

    xp = jnp.pad(x, ((0, NPAD - NN), (0, 0)))
    batchp = jnp.pad(batch.astype(jnp.int32), (0, NPAD - NN),
                     constant_values=NG).reshape(1, NPAD)

    def row(v):
        return v.reshape(1, -1)

    ones_g = jnp.zeros((NPAD, DD), jnp.float32).at[:NN].set(1.0)
    degp = _agg(ones_g, srcp, dstp)
    m1 = _mm(xp, params['W1'])
    g1, dinv = _scale(degp[0, :, 0:1], degp[1, :, 0:1], m1)

    a1 = _agg(g1, srcp, dstp)
    g2 = _norm(a1[0], a1[1], g1, dinv, row(params['b1']),
               row(params['alpha1']), row(params['gamma1']),
               row(params['beta1']), params['W2'])

    a2 = _agg(g2, srcp, dstp)
    g3 = _norm(a2[0], a2[1], g2, dinv, row(params['b2']),
               row(params['alpha2']), row(params['gamma2']),
               row(params['beta2']), params['W3'])

    a3 = _agg(g3, srcp, dstp)
    head = _make_head(params['Wh1'].shape[1], params['Wh2'].shape[1])
    return head(a3[0], a3[1], g3, dinv, row(params['b3']), batchp,
                params['Wh1'], row(params['bh1']),
                params['Wh2'], row(params['bh2']))
